# Initial kernel scaffold; baseline (speedup 1.0000x reference)
import jax
import jax.numpy as jnp
from jax.experimental import pallas as pl


def kernel(x, edge_index, edge_attr, fc1_w, fc1_b, kw1, kb1, kw2, kb2, kw3, kb3, root, bias, fc2_w, fc2_b):
    raise NotImplementedError("write your pallas kernel here")


# speedup vs baseline: 1.4184x; 1.4184x over previous
import functools

import jax
import jax.numpy as jnp
from jax import lax
from jax.experimental import pallas as pl
from jax.experimental.pallas import tpu as pltpu
from jax.experimental.pallas import tpu_sc as plsc

N = 10000
E = 160000
W = 16
KW = 128
KO = W * W
DEPTH = 4

NC = 2
NS = 16
NT = NC * NS
L = 16

ET = 5120
CH = 40
EP = NT * ET
NP = 10112
RT = NP // NS
DUMP = N

EB = 512

_mesh = plsc.VectorSubcoreMesh(core_axis_name="c", subcore_axis_name="s")
_f32 = jnp.float32
_i32 = jnp.int32


def _wid():
    return lax.axis_index("c") * NS + lax.axis_index("s")


@functools.partial(
    pl.kernel,
    out_type=(jax.ShapeDtypeStruct((EP,), _f32), jax.ShapeDtypeStruct((EP,), _f32)),
    mesh=_mesh,
    compiler_params=pltpu.CompilerParams(needs_layout_passes=False, use_tc_tiling_on_sc=False),
    scratch_types=[
        pltpu.VMEM((N,), _f32),
        pltpu.VMEM((ET,), _i32),
        pltpu.VMEM((ET,), _i32),
        pltpu.VMEM((ET,), _f32),
        pltpu.VMEM((ET,), _f32),
    ],
)
def _sc_endpoint_gather(x_hbm, src_hbm, dst_hbm, xs_hbm, xd_hbm,
                        x_v, s_v, d_v, xs_v, xd_v):
    base = _wid() * ET
    pltpu.sync_copy(x_hbm, x_v)
    pltpu.sync_copy(src_hbm.at[pl.ds(base, ET)], s_v)
    pltpu.sync_copy(dst_hbm.at[pl.ds(base, ET)], d_v)

    def body(i, carry):
        o = i * L
        xs_v[pl.ds(o, L)] = plsc.load_gather(x_v, [s_v[pl.ds(o, L)]])
        xd_v[pl.ds(o, L)] = plsc.load_gather(x_v, [d_v[pl.ds(o, L)]])
        return carry

    lax.fori_loop(0, ET // L, body, 0)
    pltpu.sync_copy(xs_v, xs_hbm.at[pl.ds(base, ET)])
    pltpu.sync_copy(xd_v, xd_hbm.at[pl.ds(base, ET)])


@functools.partial(
    pl.kernel,
    out_type=jax.ShapeDtypeStruct((EP, W), _f32),
    mesh=_mesh,
    compiler_params=pltpu.CompilerParams(needs_layout_passes=False, use_tc_tiling_on_sc=False),
    scratch_types=[
        pltpu.VMEM((ET,), _i32),
        pltpu.VMEM((ET, W), _f32),
        pltpu.SemaphoreType.DMA,
    ],
)
def _sc_gather_rows(h_hbm, src_hbm, xj_hbm, idx_v, rows_v, sem):
    base = _wid() * ET
    pltpu.sync_copy(src_hbm.at[pl.ds(base, ET)], idx_v)
    pltpu.async_copy(h_hbm.at[idx_v], rows_v, sem).wait()
    pltpu.sync_copy(rows_v, xj_hbm.at[pl.ds(base, ET)])


@functools.partial(
    pl.kernel,
    out_type=jax.ShapeDtypeStruct((NC, NP, W), _f32),
    mesh=_mesh,
    compiler_params=pltpu.CompilerParams(needs_layout_passes=False, use_tc_tiling_on_sc=False),
    scratch_types=[
        pltpu.VMEM((CH, 128), _i32),
        pltpu.VMEM((ET, W), _f32),
        pltpu.VMEM((RT, W), _f32),
        pltpu.VMEM_SHARED((NP, W), _f32),
    ],
)
def _sc_scatter_add(msg_hbm, dst3_hbm, zero_hbm, out_hbm,
                    idx_v, msg_v, buf_v, aggr_s):
    c = lax.axis_index("c")
    s = lax.axis_index("s")
    w = c * NS + s
    pltpu.sync_copy(zero_hbm, buf_v)
    pltpu.sync_copy(buf_v, aggr_s.at[pl.ds(s * RT, RT)])
    pltpu.sync_copy(dst3_hbm.at[w], idx_v)
    pltpu.sync_copy(msg_hbm.at[pl.ds(w * ET, ET)], msg_v)
    plsc.subcore_barrier()

    def body(j, carry):
        pltpu.sync_copy(msg_v.at[pl.ds(j * 128, 128)], aggr_s.at[idx_v.at[j]],
                        add=True)
        return carry

    lax.fori_loop(0, CH, body, 0)
    plsc.subcore_barrier()
    pltpu.sync_copy(aggr_s.at[pl.ds(s * RT, RT)], buf_v)
    pltpu.sync_copy(buf_v, out_hbm.at[c, pl.ds(s * RT, RT)])


@functools.partial(
    pl.kernel,
    out_type=jax.ShapeDtypeStruct((NC, NP, W), _f32),
    mesh=_mesh,
    compiler_params=pltpu.CompilerParams(needs_layout_passes=False, use_tc_tiling_on_sc=False),
    scratch_types=[
        pltpu.VMEM((CH, 128), _i32),
        pltpu.VMEM((128, W), _f32),
        pltpu.VMEM((RT, W), _f32),
        pltpu.VMEM_SHARED((NP, W), _f32),
    ],
)
def _sc_degree_count(dst3_hbm, zero_hbm, ones_hbm, out_hbm,
                     idx_v, one_v, buf_v, cnt_s):
    c = lax.axis_index("c")
    s = lax.axis_index("s")
    w = c * NS + s
    pltpu.sync_copy(zero_hbm, buf_v)
    pltpu.sync_copy(buf_v, cnt_s.at[pl.ds(s * RT, RT)])
    pltpu.sync_copy(ones_hbm, one_v)
    pltpu.sync_copy(dst3_hbm.at[w], idx_v)
    plsc.subcore_barrier()

    def body(j, carry):
        pltpu.sync_copy(one_v, cnt_s.at[idx_v.at[j]], add=True)
        return carry

    lax.fori_loop(0, CH, body, 0)
    plsc.subcore_barrier()
    pltpu.sync_copy(cnt_s.at[pl.ds(s * RT, RT)], buf_v)
    pltpu.sync_copy(buf_v, out_hbm.at[c, pl.ds(s * RT, RT)])


def _mlp_body(ea_ref, xs_ref, xd_ref, w2_ref, r3_ref, r4_ref, b1_ref,
              kw2t_ref, kb2_ref, o_ref):
    k1 = jnp.dot(ea_ref[...], w2_ref[...], preferred_element_type=_f32)
    k1 = k1 + xs_ref[...] * r3_ref[...]
    k1 = k1 + xd_ref[...] * r4_ref[...]
    k1 = jnp.maximum(k1 + b1_ref[...], 0.0)
    k2 = jnp.dot(k1, kw2t_ref[...], preferred_element_type=_f32) + kb2_ref[...]
    o_ref[...] = jnp.maximum(k2, 0.0)


def _tc_mlp(ea, xs1, xd1, w2, r3, r4, b1, kw2t, kb2):
    grid = (EP // EB,)
    return pl.pallas_call(
        _mlp_body,
        grid=grid,
        in_specs=[
            pl.BlockSpec((EB, 2), lambda i: (i, 0)),
            pl.BlockSpec((EB, 1), lambda i: (i, 0)),
            pl.BlockSpec((EB, 1), lambda i: (i, 0)),
            pl.BlockSpec((2, KW), lambda i: (0, 0)),
            pl.BlockSpec((1, KW), lambda i: (0, 0)),
            pl.BlockSpec((1, KW), lambda i: (0, 0)),
            pl.BlockSpec((1, KW), lambda i: (0, 0)),
            pl.BlockSpec((KW, KW), lambda i: (0, 0)),
            pl.BlockSpec((1, KW), lambda i: (0, 0)),
        ],
        out_specs=pl.BlockSpec((EB, KW), lambda i: (i, 0)),
        out_shape=jax.ShapeDtypeStruct((EP, KW), _f32),
    )(ea, xs1, xd1, w2, r3, r4, b1, kw2t, kb2)


def _msg_body(r2_ref, xj_ref, kw3t_ref, kb3_ref, o_ref):
    k = jnp.dot(r2_ref[...], kw3t_ref[...], preferred_element_type=_f32)
    k = k + kb3_ref[...]
    xj = xj_ref[...]
    acc = xj[:, 0:1] * k[:, 0:W]
    for i in range(1, W):
        acc = acc + xj[:, i:i + 1] * k[:, i * W:(i + 1) * W]
    o_ref[...] = acc


def _tc_msg(r2, xj, kw3t, kb3):
    grid = (EP // EB,)
    return pl.pallas_call(
        _msg_body,
        grid=grid,
        in_specs=[
            pl.BlockSpec((EB, KW), lambda i: (i, 0)),
            pl.BlockSpec((EB, W), lambda i: (i, 0)),
            pl.BlockSpec((KW, KO), lambda i: (0, 0)),
            pl.BlockSpec((1, KO), lambda i: (0, 0)),
        ],
        out_specs=pl.BlockSpec((EB, W), lambda i: (i, 0)),
        out_shape=jax.ShapeDtypeStruct((EP, W), _f32),
    )(r2, xj, kw3t, kb3)


def _init_body(x_ref, w_ref, b_ref, o_ref):
    o_ref[...] = x_ref[...] * w_ref[...] + b_ref[...]


def _tc_init(xp1, w, b):
    return pl.pallas_call(
        _init_body,
        out_shape=jax.ShapeDtypeStruct((NP, W), _f32),
    )(xp1, w, b)


def _upd_body(a_ref, c_ref, h_ref, root_ref, bias_ref, f2w_ref, f2b_ref,
              hn_ref, y_ref):
    a = a_ref[0] + a_ref[1]
    cnt = c_ref[0] + c_ref[1]
    denom = jnp.maximum(cnt, 1.0)
    hr = jnp.dot(h_ref[...], root_ref[...], preferred_element_type=_f32)
    hn = jnp.maximum(a / denom + hr + bias_ref[...], 0.0)
    hn_ref[...] = hn
    y_ref[...] = jnp.sum(hn * f2w_ref[...], axis=1, keepdims=True) + f2b_ref[...]


def _tc_update(aggr, cnt, h, root, bias1, f2w, f2b):
    return pl.pallas_call(
        _upd_body,
        out_shape=(
            jax.ShapeDtypeStruct((NP, W), _f32),
            jax.ShapeDtypeStruct((NP, 1), _f32),
        ),
    )(aggr, cnt, h, root, bias1, f2w, f2b)


def kernel(x, edge_index, edge_attr, fc1_w, fc1_b, kw1, kb1, kw2, kb2,
           kw3, kb3, root, bias, fc2_w, fc2_b):
    src = edge_index[0].astype(_i32)
    dst = edge_index[1].astype(_i32)
    pad = EP - E
    src_p = jnp.concatenate([src, jnp.zeros((pad,), _i32)])
    dst_p = jnp.concatenate([dst, jnp.full((pad,), DUMP, _i32)])
    dst3 = dst_p.reshape(NT, CH, 128)
    ea_p = jnp.pad(edge_attr, ((0, pad), (0, 0)))
    x_p = jnp.pad(x, (0, NP - N))

    zero_rt = jnp.zeros((RT, W), _f32)
    ones_blk = jnp.ones((128, W), _f32)

    xs, xd = _sc_endpoint_gather(x, src_p, dst_p)

    w2 = kw1[:, 0:2].T
    r3 = kw1[:, 2:3].T
    r4 = kw1[:, 3:4].T
    r2 = _tc_mlp(ea_p, xs[:, None], xd[:, None], w2, r3, r4, kb1[None, :],
                 kw2.T, kb2[None, :])

    h = _tc_init(x_p[:, None], fc1_w.T, fc1_b[None, :])

    cnt = _sc_degree_count(dst3, zero_rt, ones_blk)

    kw3t = kw3.T
    kb31 = kb3[None, :]
    bias1 = bias[None, :]
    f2b = fc2_b[None, :]

    y = None
    for _ in range(DEPTH):
        xj = _sc_gather_rows(h, src_p)
        msg = _tc_msg(r2, xj, kw3t, kb31)
        aggr = _sc_scatter_add(msg, dst3, zero_rt)
        h, y = _tc_update(aggr, cnt, h, root, bias1, fc2_w, f2b)

    return y[:N, 0]


# speedup vs baseline: 2.5383x; 1.7895x over previous
import functools

import jax
import jax.numpy as jnp
from jax import lax
from jax.experimental import pallas as pl
from jax.experimental.pallas import tpu as pltpu
from jax.experimental.pallas import tpu_sc as plsc

N = 10000
E = 160000
W = 16
KW = 128
KO = W * W
DEPTH = 4

NC = 2
NS = 16
NT = NC * NS
L = 16

ET = 5120
CH = 40
EP = NT * ET
NP = 10112
RT = NP // NS
DUMP = N

EB = 512

_mesh = plsc.VectorSubcoreMesh(core_axis_name="c", subcore_axis_name="s")
_f32 = jnp.float32
_i32 = jnp.int32


def _wid():
    return lax.axis_index("c") * NS + lax.axis_index("s")


@functools.partial(
    pl.kernel,
    out_type=(jax.ShapeDtypeStruct((EP,), _f32), jax.ShapeDtypeStruct((EP,), _f32)),
    mesh=_mesh,
    compiler_params=pltpu.CompilerParams(needs_layout_passes=False, use_tc_tiling_on_sc=False),
    scratch_types=[
        pltpu.VMEM((N,), _f32),
        pltpu.VMEM((ET,), _i32),
        pltpu.VMEM((ET,), _i32),
        pltpu.VMEM((ET,), _f32),
        pltpu.VMEM((ET,), _f32),
    ],
)
def _sc_endpoint_gather(x_hbm, src_hbm, dst_hbm, xs_hbm, xd_hbm,
                        x_v, s_v, d_v, xs_v, xd_v):
    base = _wid() * ET
    pltpu.sync_copy(x_hbm, x_v)
    pltpu.sync_copy(src_hbm.at[pl.ds(base, ET)], s_v)
    pltpu.sync_copy(dst_hbm.at[pl.ds(base, ET)], d_v)

    def body(i, carry):
        o = i * L
        xs_v[pl.ds(o, L)] = plsc.load_gather(x_v, [s_v[pl.ds(o, L)]])
        xd_v[pl.ds(o, L)] = plsc.load_gather(x_v, [d_v[pl.ds(o, L)]])
        return carry

    lax.fori_loop(0, ET // L, body, 0)
    pltpu.sync_copy(xs_v, xs_hbm.at[pl.ds(base, ET)])
    pltpu.sync_copy(xd_v, xd_hbm.at[pl.ds(base, ET)])


@functools.partial(
    pl.kernel,
    out_type=jax.ShapeDtypeStruct((EP, W), _f32),
    mesh=_mesh,
    compiler_params=pltpu.CompilerParams(needs_layout_passes=False, use_tc_tiling_on_sc=False),
    scratch_types=[
        pltpu.VMEM((ET,), _i32),
        pltpu.VMEM((ET, W), _f32),
        pltpu.SemaphoreType.DMA,
    ],
)
def _sc_gather_rows(h_hbm, src_hbm, xj_hbm, idx_v, rows_v, sem):
    base = _wid() * ET
    pltpu.sync_copy(src_hbm.at[pl.ds(base, ET)], idx_v)
    pltpu.async_copy(h_hbm.at[idx_v], rows_v, sem).wait()
    pltpu.sync_copy(rows_v, xj_hbm.at[pl.ds(base, ET)])


@functools.partial(
    pl.kernel,
    out_type=jax.ShapeDtypeStruct((NC, NP, W), _f32),
    mesh=_mesh,
    compiler_params=pltpu.CompilerParams(needs_layout_passes=False, use_tc_tiling_on_sc=False),
    scratch_types=[
        pltpu.VMEM((CH, 128), _i32),
        pltpu.VMEM((ET, W), _f32),
        pltpu.VMEM((RT, W), _f32),
        pltpu.VMEM_SHARED((NP, W), _f32),
    ],
)
def _sc_scatter_add(msg_hbm, dst3_hbm, zero_hbm, out_hbm,
                    idx_v, msg_v, buf_v, aggr_s):
    c = lax.axis_index("c")
    s = lax.axis_index("s")
    w = c * NS + s
    pltpu.sync_copy(zero_hbm, buf_v)
    pltpu.sync_copy(buf_v, aggr_s.at[pl.ds(s * RT, RT)])
    pltpu.sync_copy(dst3_hbm.at[w], idx_v)
    pltpu.sync_copy(msg_hbm.at[pl.ds(w * ET, ET)], msg_v)
    plsc.subcore_barrier()

    def body(j, carry):
        pltpu.sync_copy(msg_v.at[pl.ds(j * 128, 128)], aggr_s.at[idx_v.at[j]],
                        add=True)
        return carry

    lax.fori_loop(0, CH, body, 0)
    plsc.subcore_barrier()
    pltpu.sync_copy(aggr_s.at[pl.ds(s * RT, RT)], buf_v)
    pltpu.sync_copy(buf_v, out_hbm.at[c, pl.ds(s * RT, RT)])


@functools.partial(
    pl.kernel,
    out_type=jax.ShapeDtypeStruct((NC, NP, W), _f32),
    mesh=_mesh,
    compiler_params=pltpu.CompilerParams(needs_layout_passes=False, use_tc_tiling_on_sc=False),
    scratch_types=[
        pltpu.VMEM((CH, 128), _i32),
        pltpu.VMEM((128, W), _f32),
        pltpu.VMEM((RT, W), _f32),
        pltpu.VMEM_SHARED((NP, W), _f32),
    ],
)
def _sc_degree_count(dst3_hbm, zero_hbm, ones_hbm, out_hbm,
                     idx_v, one_v, buf_v, cnt_s):
    c = lax.axis_index("c")
    s = lax.axis_index("s")
    w = c * NS + s
    pltpu.sync_copy(zero_hbm, buf_v)
    pltpu.sync_copy(buf_v, cnt_s.at[pl.ds(s * RT, RT)])
    pltpu.sync_copy(ones_hbm, one_v)
    pltpu.sync_copy(dst3_hbm.at[w], idx_v)
    plsc.subcore_barrier()

    def body(j, carry):
        pltpu.sync_copy(one_v, cnt_s.at[idx_v.at[j]], add=True)
        return carry

    lax.fori_loop(0, CH, body, 0)
    plsc.subcore_barrier()
    pltpu.sync_copy(cnt_s.at[pl.ds(s * RT, RT)], buf_v)
    pltpu.sync_copy(buf_v, out_hbm.at[c, pl.ds(s * RT, RT)])


def _mlp_body(ea_ref, xs_ref, xd_ref, w2_ref, r3_ref, r4_ref, b1_ref,
              kw2t_ref, kb2_ref, o_ref):
    k1 = jnp.dot(ea_ref[...], w2_ref[...], preferred_element_type=_f32)
    k1 = k1 + xs_ref[...] * r3_ref[...]
    k1 = k1 + xd_ref[...] * r4_ref[...]
    k1 = jnp.maximum(k1 + b1_ref[...], 0.0)
    k2 = jnp.dot(k1, kw2t_ref[...], preferred_element_type=_f32) + kb2_ref[...]
    o_ref[...] = jnp.maximum(k2, 0.0)


def _tc_mlp(ea, xs1, xd1, w2, r3, r4, b1, kw2t, kb2):
    grid = (EP // EB,)
    return pl.pallas_call(
        _mlp_body,
        grid=grid,
        in_specs=[
            pl.BlockSpec((EB, 2), lambda i: (i, 0)),
            pl.BlockSpec((EB, 1), lambda i: (i, 0)),
            pl.BlockSpec((EB, 1), lambda i: (i, 0)),
            pl.BlockSpec((2, KW), lambda i: (0, 0)),
            pl.BlockSpec((1, KW), lambda i: (0, 0)),
            pl.BlockSpec((1, KW), lambda i: (0, 0)),
            pl.BlockSpec((1, KW), lambda i: (0, 0)),
            pl.BlockSpec((KW, KW), lambda i: (0, 0)),
            pl.BlockSpec((1, KW), lambda i: (0, 0)),
        ],
        out_specs=pl.BlockSpec((EB, KW), lambda i: (i, 0)),
        out_shape=jax.ShapeDtypeStruct((EP, KW), _f32),
    )(ea, xs1, xd1, w2, r3, r4, b1, kw2t, kb2)


def _msg_body(r2_ref, xj_ref, kw3t_ref, kb3_ref, rsel_ref, ssel_ref, o_ref):
    k = jnp.dot(r2_ref[...], kw3t_ref[...], preferred_element_type=_f32)
    k = k + kb3_ref[...]
    xrep = jnp.dot(xj_ref[...], rsel_ref[...], preferred_element_type=_f32)
    o_ref[...] = jnp.dot(xrep * k, ssel_ref[...], preferred_element_type=_f32)


def _tc_msg(r2, xj, kw3t, kb3, rsel, ssel):
    grid = (EP // EB,)
    return pl.pallas_call(
        _msg_body,
        grid=grid,
        in_specs=[
            pl.BlockSpec((EB, KW), lambda i: (i, 0)),
            pl.BlockSpec((EB, W), lambda i: (i, 0)),
            pl.BlockSpec((KW, KO), lambda i: (0, 0)),
            pl.BlockSpec((1, KO), lambda i: (0, 0)),
            pl.BlockSpec((W, KO), lambda i: (0, 0)),
            pl.BlockSpec((KO, W), lambda i: (0, 0)),
        ],
        out_specs=pl.BlockSpec((EB, W), lambda i: (i, 0)),
        out_shape=jax.ShapeDtypeStruct((EP, W), _f32),
    )(r2, xj, kw3t, kb3, rsel, ssel)


def _init_body(x_ref, w_ref, b_ref, o_ref):
    o_ref[...] = x_ref[...] * w_ref[...] + b_ref[...]


def _tc_init(xp1, w, b):
    return pl.pallas_call(
        _init_body,
        out_shape=jax.ShapeDtypeStruct((NP, W), _f32),
    )(xp1, w, b)


def _upd_body(a_ref, c_ref, h_ref, root_ref, bias_ref, f2w_ref, f2b_ref,
              hn_ref, y_ref):
    a = a_ref[0] + a_ref[1]
    cnt = c_ref[0] + c_ref[1]
    denom = jnp.maximum(cnt, 1.0)
    hr = jnp.dot(h_ref[...], root_ref[...], preferred_element_type=_f32)
    hn = jnp.maximum(a / denom + hr + bias_ref[...], 0.0)
    hn_ref[...] = hn
    y_ref[...] = jnp.sum(hn * f2w_ref[...], axis=1, keepdims=True) + f2b_ref[...]


def _tc_update(aggr, cnt, h, root, bias1, f2w, f2b):
    return pl.pallas_call(
        _upd_body,
        out_shape=(
            jax.ShapeDtypeStruct((NP, W), _f32),
            jax.ShapeDtypeStruct((NP, 1), _f32),
        ),
    )(aggr, cnt, h, root, bias1, f2w, f2b)


def kernel(x, edge_index, edge_attr, fc1_w, fc1_b, kw1, kb1, kw2, kb2,
           kw3, kb3, root, bias, fc2_w, fc2_b):
    src = edge_index[0].astype(_i32)
    dst = edge_index[1].astype(_i32)
    pad = EP - E
    src_p = jnp.concatenate([src, jnp.zeros((pad,), _i32)])
    dst_p = jnp.concatenate([dst, jnp.full((pad,), DUMP, _i32)])
    dst3 = dst_p.reshape(NT, CH, 128)
    ea_p = jnp.pad(edge_attr, ((0, pad), (0, 0)))
    x_p = jnp.pad(x, (0, NP - N))

    zero_rt = jnp.zeros((RT, W), _f32)
    ones_blk = jnp.ones((128, W), _f32)

    xs, xd = _sc_endpoint_gather(x, src_p, dst_p)

    w2 = kw1[:, 0:2].T
    r3 = kw1[:, 2:3].T
    r4 = kw1[:, 3:4].T
    r2 = _tc_mlp(ea_p, xs[:, None], xd[:, None], w2, r3, r4, kb1[None, :],
                 kw2.T, kb2[None, :])

    h = _tc_init(x_p[:, None], fc1_w.T, fc1_b[None, :])

    cnt = _sc_degree_count(dst3, zero_rt, ones_blk)

    kw3t = kw3.T
    kb31 = kb3[None, :]
    lane = jnp.arange(KO, dtype=_i32)
    rsel = (lane[None, :] // W == jnp.arange(W, dtype=_i32)[:, None]).astype(_f32)
    ssel = (lane[:, None] % W == jnp.arange(W, dtype=_i32)[None, :]).astype(_f32)
    bias1 = bias[None, :]
    f2b = fc2_b[None, :]

    y = None
    for _ in range(DEPTH):
        xj = _sc_gather_rows(h, src_p)
        msg = _tc_msg(r2, xj, kw3t, kb31, rsel, ssel)
        aggr = _sc_scatter_add(msg, dst3, zero_rt)
        h, y = _tc_update(aggr, cnt, h, root, bias1, fc2_w, f2b)

    return y[:N, 0]


# speedup vs baseline: 3.2315x; 1.2731x over previous
import functools

import jax
import jax.numpy as jnp
from jax import lax
from jax.experimental import pallas as pl
from jax.experimental.pallas import tpu as pltpu
from jax.experimental.pallas import tpu_sc as plsc

N = 10000
E = 160000
W = 16
KW = 128
KO = W * W
DEPTH = 4

NC = 2
NS = 16
NT = NC * NS
L = 16

ET = 5120
CH = 40
EP = NT * ET
NP = 10112
RT = NP // NS
DUMP = N

EB = 1024

_mesh = plsc.VectorSubcoreMesh(core_axis_name="c", subcore_axis_name="s")
_f32 = jnp.float32
_bf16 = jnp.bfloat16
_i32 = jnp.int32


def _wid():
    return lax.axis_index("c") * NS + lax.axis_index("s")


@functools.partial(
    pl.kernel,
    out_type=(jax.ShapeDtypeStruct((EP,), _f32), jax.ShapeDtypeStruct((EP,), _f32)),
    mesh=_mesh,
    compiler_params=pltpu.CompilerParams(needs_layout_passes=False, use_tc_tiling_on_sc=False),
    scratch_types=[
        pltpu.VMEM((N,), _f32),
        pltpu.VMEM((ET,), _i32),
        pltpu.VMEM((ET,), _i32),
        pltpu.VMEM((ET,), _f32),
        pltpu.VMEM((ET,), _f32),
    ],
)
def _sc_endpoint_gather(x_hbm, src_hbm, dst_hbm, xs_hbm, xd_hbm,
                        x_v, s_v, d_v, xs_v, xd_v):
    base = _wid() * ET
    pltpu.sync_copy(x_hbm, x_v)
    pltpu.sync_copy(src_hbm.at[pl.ds(base, ET)], s_v)
    pltpu.sync_copy(dst_hbm.at[pl.ds(base, ET)], d_v)

    def body(i, carry):
        o = i * L
        xs_v[pl.ds(o, L)] = plsc.load_gather(x_v, [s_v[pl.ds(o, L)]])
        xd_v[pl.ds(o, L)] = plsc.load_gather(x_v, [d_v[pl.ds(o, L)]])
        return carry

    lax.fori_loop(0, ET // L, body, 0)
    pltpu.sync_copy(xs_v, xs_hbm.at[pl.ds(base, ET)])
    pltpu.sync_copy(xd_v, xd_hbm.at[pl.ds(base, ET)])


@functools.partial(
    pl.kernel,
    out_type=jax.ShapeDtypeStruct((EP, W), _f32),
    mesh=_mesh,
    compiler_params=pltpu.CompilerParams(needs_layout_passes=False, use_tc_tiling_on_sc=False),
    scratch_types=[
        pltpu.VMEM((ET,), _i32),
        pltpu.VMEM((ET, W), _f32),
        pltpu.SemaphoreType.DMA,
    ],
)
def _sc_gather_rows(h_hbm, src_hbm, xj_hbm, idx_v, rows_v, sem):
    base = _wid() * ET
    pltpu.sync_copy(src_hbm.at[pl.ds(base, ET)], idx_v)
    pltpu.async_copy(h_hbm.at[idx_v], rows_v, sem).wait()
    pltpu.sync_copy(rows_v, xj_hbm.at[pl.ds(base, ET)])


@functools.partial(
    pl.kernel,
    out_type=jax.ShapeDtypeStruct((NC, NP, W), _f32),
    mesh=_mesh,
    compiler_params=pltpu.CompilerParams(needs_layout_passes=False, use_tc_tiling_on_sc=False),
    scratch_types=[
        pltpu.VMEM((CH, 128), _i32),
        pltpu.VMEM((ET, W), _f32),
        pltpu.VMEM((RT, W), _f32),
        pltpu.VMEM_SHARED((NP, W), _f32),
    ],
)
def _sc_scatter_add(msg_hbm, dst3_hbm, zero_hbm, out_hbm,
                    idx_v, msg_v, buf_v, aggr_s):
    c = lax.axis_index("c")
    s = lax.axis_index("s")
    w = c * NS + s
    pltpu.sync_copy(zero_hbm, buf_v)
    pltpu.sync_copy(buf_v, aggr_s.at[pl.ds(s * RT, RT)])
    pltpu.sync_copy(dst3_hbm.at[w], idx_v)
    pltpu.sync_copy(msg_hbm.at[pl.ds(w * ET, ET)], msg_v)
    plsc.subcore_barrier()

    def body(j, carry):
        pltpu.sync_copy(msg_v.at[pl.ds(j * 128, 128)], aggr_s.at[idx_v.at[j]],
                        add=True)
        return carry

    lax.fori_loop(0, CH, body, 0)
    plsc.subcore_barrier()
    pltpu.sync_copy(aggr_s.at[pl.ds(s * RT, RT)], buf_v)
    pltpu.sync_copy(buf_v, out_hbm.at[c, pl.ds(s * RT, RT)])


@functools.partial(
    pl.kernel,
    out_type=jax.ShapeDtypeStruct((NC, NP, W), _f32),
    mesh=_mesh,
    compiler_params=pltpu.CompilerParams(needs_layout_passes=False, use_tc_tiling_on_sc=False),
    scratch_types=[
        pltpu.VMEM((CH, 128), _i32),
        pltpu.VMEM((128, W), _f32),
        pltpu.VMEM((RT, W), _f32),
        pltpu.VMEM_SHARED((NP, W), _f32),
    ],
)
def _sc_degree_count(dst3_hbm, zero_hbm, ones_hbm, out_hbm,
                     idx_v, one_v, buf_v, cnt_s):
    c = lax.axis_index("c")
    s = lax.axis_index("s")
    w = c * NS + s
    pltpu.sync_copy(zero_hbm, buf_v)
    pltpu.sync_copy(buf_v, cnt_s.at[pl.ds(s * RT, RT)])
    pltpu.sync_copy(ones_hbm, one_v)
    pltpu.sync_copy(dst3_hbm.at[w], idx_v)
    plsc.subcore_barrier()

    def body(j, carry):
        pltpu.sync_copy(one_v, cnt_s.at[idx_v.at[j]], add=True)
        return carry

    lax.fori_loop(0, CH, body, 0)
    plsc.subcore_barrier()
    pltpu.sync_copy(cnt_s.at[pl.ds(s * RT, RT)], buf_v)
    pltpu.sync_copy(buf_v, out_hbm.at[c, pl.ds(s * RT, RT)])


def _mlp_body(ea_ref, xs_ref, xd_ref, w2_ref, r3_ref, r4_ref, b1_ref,
              kw2t_ref, kb2_ref, o_ref):
    k1 = jnp.dot(ea_ref[...], w2_ref[...], preferred_element_type=_f32)
    k1 = k1 + xs_ref[...] * r3_ref[...]
    k1 = k1 + xd_ref[...] * r4_ref[...]
    k1 = jnp.maximum(k1 + b1_ref[...], 0.0).astype(_bf16)
    k2 = jnp.dot(k1, kw2t_ref[...], preferred_element_type=_f32) + kb2_ref[...]
    o_ref[...] = jnp.maximum(k2, 0.0).astype(_bf16)


def _tc_mlp(ea, xs1, xd1, w2, r3, r4, b1, kw2t, kb2):
    grid = (EP // EB,)
    return pl.pallas_call(
        _mlp_body,
        grid=grid,
        in_specs=[
            pl.BlockSpec((EB, 2), lambda i: (i, 0)),
            pl.BlockSpec((EB, 1), lambda i: (i, 0)),
            pl.BlockSpec((EB, 1), lambda i: (i, 0)),
            pl.BlockSpec((2, KW), lambda i: (0, 0)),
            pl.BlockSpec((1, KW), lambda i: (0, 0)),
            pl.BlockSpec((1, KW), lambda i: (0, 0)),
            pl.BlockSpec((1, KW), lambda i: (0, 0)),
            pl.BlockSpec((KW, KW), lambda i: (0, 0)),
            pl.BlockSpec((1, KW), lambda i: (0, 0)),
        ],
        out_specs=pl.BlockSpec((EB, KW), lambda i: (i, 0)),
        out_shape=jax.ShapeDtypeStruct((EP, KW), _bf16),
    )(ea, xs1, xd1, w2, r3, r4, b1, kw2t, kb2)


def _msg_body(r2_ref, xj_ref, kw3t_ref, kb3_ref, rsel_ref, ssel_ref, o_ref):
    k = jnp.dot(r2_ref[...], kw3t_ref[...], preferred_element_type=_f32)
    k = k + kb3_ref[...]
    xrep = jnp.dot(xj_ref[...], rsel_ref[...], preferred_element_type=_f32)
    o_ref[...] = jnp.dot(xrep * k, ssel_ref[...], preferred_element_type=_f32)


def _tc_msg(r2, xj, kw3t, kb3, rsel, ssel):
    grid = (EP // EB,)
    return pl.pallas_call(
        _msg_body,
        grid=grid,
        in_specs=[
            pl.BlockSpec((EB, KW), lambda i: (i, 0)),
            pl.BlockSpec((EB, W), lambda i: (i, 0)),
            pl.BlockSpec((KW, KO), lambda i: (0, 0)),
            pl.BlockSpec((1, KO), lambda i: (0, 0)),
            pl.BlockSpec((W, KO), lambda i: (0, 0)),
            pl.BlockSpec((KO, W), lambda i: (0, 0)),
        ],
        out_specs=pl.BlockSpec((EB, W), lambda i: (i, 0)),
        out_shape=jax.ShapeDtypeStruct((EP, W), _f32),
    )(r2, xj, kw3t, kb3, rsel, ssel)


def _init_body(x_ref, w_ref, b_ref, o_ref):
    o_ref[...] = x_ref[...] * w_ref[...] + b_ref[...]


def _tc_init(xp1, w, b):
    return pl.pallas_call(
        _init_body,
        out_shape=jax.ShapeDtypeStruct((NP, W), _f32),
    )(xp1, w, b)


def _upd_body(a_ref, c_ref, h_ref, root_ref, bias_ref, f2w_ref, f2b_ref,
              hn_ref, y_ref):
    a = a_ref[0] + a_ref[1]
    cnt = c_ref[0] + c_ref[1]
    denom = jnp.maximum(cnt, 1.0)
    hr = jnp.dot(h_ref[...], root_ref[...], preferred_element_type=_f32)
    hn = jnp.maximum(a / denom + hr + bias_ref[...], 0.0)
    hn_ref[...] = hn
    y_ref[...] = jnp.sum(hn * f2w_ref[...], axis=1, keepdims=True) + f2b_ref[...]


def _tc_update(aggr, cnt, h, root, bias1, f2w, f2b):
    return pl.pallas_call(
        _upd_body,
        out_shape=(
            jax.ShapeDtypeStruct((NP, W), _f32),
            jax.ShapeDtypeStruct((NP, 1), _f32),
        ),
    )(aggr, cnt, h, root, bias1, f2w, f2b)


def kernel(x, edge_index, edge_attr, fc1_w, fc1_b, kw1, kb1, kw2, kb2,
           kw3, kb3, root, bias, fc2_w, fc2_b):
    src = edge_index[0].astype(_i32)
    dst = edge_index[1].astype(_i32)
    pad = EP - E
    src_p = jnp.concatenate([src, jnp.zeros((pad,), _i32)])
    dst_p = jnp.concatenate([dst, jnp.full((pad,), DUMP, _i32)])
    dst3 = dst_p.reshape(NT, CH, 128)
    ea_p = jnp.pad(edge_attr, ((0, pad), (0, 0)))
    x_p = jnp.pad(x, (0, NP - N))

    zero_rt = jnp.zeros((RT, W), _f32)
    ones_blk = jnp.ones((128, W), _f32)

    xs, xd = _sc_endpoint_gather(x, src_p, dst_p)

    w2 = kw1[:, 0:2].T
    r3 = kw1[:, 2:3].T
    r4 = kw1[:, 3:4].T
    r2 = _tc_mlp(ea_p, xs[:, None], xd[:, None], w2, r3, r4, kb1[None, :],
                 kw2.T.astype(_bf16), kb2[None, :])

    h = _tc_init(x_p[:, None], fc1_w.T, fc1_b[None, :])

    cnt = _sc_degree_count(dst3, zero_rt, ones_blk)

    kw3t = kw3.T.astype(_bf16)
    kb31 = kb3[None, :]
    lane = jnp.arange(KO, dtype=_i32)
    rsel = (lane[None, :] // W == jnp.arange(W, dtype=_i32)[:, None]).astype(_f32)
    ssel = (lane[:, None] % W == jnp.arange(W, dtype=_i32)[None, :]).astype(_f32)
    bias1 = bias[None, :]
    f2b = fc2_b[None, :]

    y = None
    for _ in range(DEPTH):
        xj = _sc_gather_rows(h, src_p)
        msg = _tc_msg(r2, xj, kw3t, kb31, rsel, ssel)
        aggr = _sc_scatter_add(msg, dst3, zero_rt)
        h, y = _tc_update(aggr, cnt, h, root, bias1, fc2_w, f2b)

    return y[:N, 0]


# speedup vs baseline: 4.3131x; 1.3347x over previous
import functools

import jax
import jax.numpy as jnp
from jax import lax
from jax.experimental import pallas as pl
from jax.experimental.pallas import tpu as pltpu
from jax.experimental.pallas import tpu_sc as plsc

N = 10000
E = 160000
W = 16
KW = 128
KO = W * W
DEPTH = 4

NC = 2
NS = 16
NT = NC * NS
L = 16

ET = 5120
CH = 40
CG = 1280
EP = NT * ET
NP = 10112
RT = NP // NS
DUMP = N

EB = 1024

_mesh = plsc.VectorSubcoreMesh(core_axis_name="c", subcore_axis_name="s")
_f32 = jnp.float32
_bf16 = jnp.bfloat16
_i32 = jnp.int32
_sc_params = pltpu.CompilerParams(
    needs_layout_passes=False, use_tc_tiling_on_sc=False)


def _wid():
    return lax.axis_index("c") * NS + lax.axis_index("s")


@functools.partial(
    pl.kernel,
    out_type=(jax.ShapeDtypeStruct((EP,), _f32), jax.ShapeDtypeStruct((EP,), _f32)),
    mesh=_mesh,
    compiler_params=_sc_params,
    scratch_types=[
        pltpu.VMEM((N,), _f32),
        pltpu.VMEM((ET,), _i32),
        pltpu.VMEM((ET,), _i32),
        pltpu.VMEM((ET,), _f32),
        pltpu.VMEM((ET,), _f32),
    ],
)
def _sc_endpoint_gather(x_hbm, src_hbm, dst_hbm, xs_hbm, xd_hbm,
                        x_v, s_v, d_v, xs_v, xd_v):
    base = _wid() * ET
    pltpu.sync_copy(x_hbm, x_v)
    pltpu.sync_copy(src_hbm.at[pl.ds(base, ET)], s_v)
    pltpu.sync_copy(dst_hbm.at[pl.ds(base, ET)], d_v)

    def body(i, carry):
        o = i * L
        xs_v[pl.ds(o, L)] = plsc.load_gather(x_v, [s_v[pl.ds(o, L)]])
        xd_v[pl.ds(o, L)] = plsc.load_gather(x_v, [d_v[pl.ds(o, L)]])
        return carry

    lax.fori_loop(0, ET // L, body, 0)
    pltpu.sync_copy(xs_v, xs_hbm.at[pl.ds(base, ET)])
    pltpu.sync_copy(xd_v, xd_hbm.at[pl.ds(base, ET)])


@functools.partial(
    pl.kernel,
    out_type=jax.ShapeDtypeStruct((W, EP), _f32),
    mesh=_mesh,
    compiler_params=_sc_params,
    scratch_types=[
        pltpu.VMEM((ET,), _i32),
        pltpu.VMEM((CG, W), _f32),
        pltpu.VMEM((W, ET), _f32),
        pltpu.SemaphoreType.DMA,
    ],
)
def _sc_gather_rows_t(h_hbm, src_hbm, xjt_hbm, idx_v, rows_v, xjt_v, sem):
    base = _wid() * ET
    pltpu.sync_copy(src_hbm.at[pl.ds(base, ET)], idx_v)
    lane = jnp.arange(L, dtype=_i32)

    def chunk(cix, carry):
        cb = cix * CG
        pltpu.async_copy(h_hbm.at[idx_v.at[pl.ds(cb, CG)]], rows_v, sem).wait()

        def grp(g, carry2):
            rb = g * L
            ridx = lane + rb
            for i in range(W):
                vec = plsc.load_gather(rows_v, [ridx, jnp.full((L,), i, _i32)])
                xjt_v[i, pl.ds(cb + rb, L)] = vec
            return carry2

        lax.fori_loop(0, CG // L, grp, 0)
        return carry

    lax.fori_loop(0, ET // CG, chunk, 0)
    for i in range(W):
        pltpu.sync_copy(xjt_v.at[i], xjt_hbm.at[i, pl.ds(base, ET)])


@functools.partial(
    pl.kernel,
    out_type=jax.ShapeDtypeStruct((NC, NP, W), _f32),
    mesh=_mesh,
    compiler_params=_sc_params,
    scratch_types=[
        pltpu.VMEM((CH, 128), _i32),
        pltpu.VMEM((W, ET), _f32),
        pltpu.VMEM((128, W), _f32),
        pltpu.VMEM((RT, W), _f32),
        pltpu.VMEM_SHARED((NP, W), _f32),
    ],
)
def _sc_scatter_add_t(msgt_hbm, dst3_hbm, zero_hbm, out_hbm,
                      idx_v, msgt_v, chunk_v, buf_v, aggr_s):
    c = lax.axis_index("c")
    s = lax.axis_index("s")
    w = c * NS + s
    base = w * ET
    pltpu.sync_copy(zero_hbm, buf_v)
    pltpu.sync_copy(buf_v, aggr_s.at[pl.ds(s * RT, RT)])
    pltpu.sync_copy(dst3_hbm.at[w], idx_v)
    for i in range(W):
        pltpu.sync_copy(msgt_hbm.at[i, pl.ds(base, ET)], msgt_v.at[i])
    plsc.subcore_barrier()
    lane = jnp.arange(L, dtype=_i32)

    def body(j, carry):
        jb = j * 128

        def grp(g, carry2):
            eb = g * L
            eidx = lane + eb
            for i in range(W):
                vec = msgt_v[i, pl.ds(jb + eb, L)]
                plsc.store_scatter(chunk_v, [eidx, jnp.full((L,), i, _i32)], vec)
            return carry2

        lax.fori_loop(0, 128 // L, grp, 0)
        pltpu.sync_copy(chunk_v, aggr_s.at[idx_v.at[j]], add=True)
        return carry

    lax.fori_loop(0, CH, body, 0)
    plsc.subcore_barrier()
    pltpu.sync_copy(aggr_s.at[pl.ds(s * RT, RT)], buf_v)
    pltpu.sync_copy(buf_v, out_hbm.at[c, pl.ds(s * RT, RT)])


@functools.partial(
    pl.kernel,
    out_type=jax.ShapeDtypeStruct((NC, NP, W), _f32),
    mesh=_mesh,
    compiler_params=_sc_params,
    scratch_types=[
        pltpu.VMEM((CH, 128), _i32),
        pltpu.VMEM((128, W), _f32),
        pltpu.VMEM((RT, W), _f32),
        pltpu.VMEM_SHARED((NP, W), _f32),
    ],
)
def _sc_degree_count(dst3_hbm, zero_hbm, ones_hbm, out_hbm,
                     idx_v, one_v, buf_v, cnt_s):
    c = lax.axis_index("c")
    s = lax.axis_index("s")
    w = c * NS + s
    pltpu.sync_copy(zero_hbm, buf_v)
    pltpu.sync_copy(buf_v, cnt_s.at[pl.ds(s * RT, RT)])
    pltpu.sync_copy(ones_hbm, one_v)
    pltpu.sync_copy(dst3_hbm.at[w], idx_v)
    plsc.subcore_barrier()

    def body(j, carry):
        pltpu.sync_copy(one_v, cnt_s.at[idx_v.at[j]], add=True)
        return carry

    lax.fori_loop(0, CH, body, 0)
    plsc.subcore_barrier()
    pltpu.sync_copy(cnt_s.at[pl.ds(s * RT, RT)], buf_v)
    pltpu.sync_copy(buf_v, out_hbm.at[c, pl.ds(s * RT, RT)])


def _mlp_body(eat_ref, kw1_ref, kb1_ref, kw2_ref, kb2_ref, o_ref):
    k1 = jnp.dot(kw1_ref[...], eat_ref[...], preferred_element_type=_f32)
    k1 = jnp.maximum(k1 + kb1_ref[...], 0.0).astype(_bf16)
    k2 = jnp.dot(kw2_ref[...], k1, preferred_element_type=_f32) + kb2_ref[...]
    o_ref[...] = jnp.maximum(k2, 0.0).astype(_bf16)


def _tc_mlp(eat, kw1, kb1c, kw2b, kb2c):
    grid = (EP // EB,)
    return pl.pallas_call(
        _mlp_body,
        grid=grid,
        in_specs=[
            pl.BlockSpec((4, EB), lambda i: (0, i)),
            pl.BlockSpec((KW, 4), lambda i: (0, 0)),
            pl.BlockSpec((KW, 1), lambda i: (0, 0)),
            pl.BlockSpec((KW, KW), lambda i: (0, 0)),
            pl.BlockSpec((KW, 1), lambda i: (0, 0)),
        ],
        out_specs=pl.BlockSpec((KW, EB), lambda i: (0, i)),
        out_shape=jax.ShapeDtypeStruct((KW, EP), _bf16),
    )(eat, kw1, kb1c, kw2b, kb2c)


def _msg_body(r2t_ref, xjt_ref, kw3_ref, kb3_ref, rsel_ref, ssel_ref, o_ref):
    k = jnp.dot(kw3_ref[...], r2t_ref[...], preferred_element_type=_f32)
    k = k + kb3_ref[...]
    xrep = jnp.dot(rsel_ref[...], xjt_ref[...], preferred_element_type=_f32)
    o_ref[...] = jnp.dot(ssel_ref[...], xrep * k, preferred_element_type=_f32)


def _tc_msg(r2t, xjt, kw3b, kb3c, rsel, ssel):
    grid = (EP // EB,)
    return pl.pallas_call(
        _msg_body,
        grid=grid,
        in_specs=[
            pl.BlockSpec((KW, EB), lambda i: (0, i)),
            pl.BlockSpec((W, EB), lambda i: (0, i)),
            pl.BlockSpec((KO, KW), lambda i: (0, 0)),
            pl.BlockSpec((KO, 1), lambda i: (0, 0)),
            pl.BlockSpec((KO, W), lambda i: (0, 0)),
            pl.BlockSpec((W, KO), lambda i: (0, 0)),
        ],
        out_specs=pl.BlockSpec((W, EB), lambda i: (0, i)),
        out_shape=jax.ShapeDtypeStruct((W, EP), _f32),
    )(r2t, xjt, kw3b, kb3c, rsel, ssel)


def _init_body(x_ref, w_ref, b_ref, o_ref):
    o_ref[...] = x_ref[...] * w_ref[...] + b_ref[...]


def _tc_init(xp1, w, b):
    return pl.pallas_call(
        _init_body,
        out_shape=jax.ShapeDtypeStruct((NP, W), _f32),
    )(xp1, w, b)


def _upd_body(a_ref, c_ref, h_ref, root_ref, bias_ref, f2w_ref, f2b_ref,
              hn_ref, y_ref):
    a = a_ref[0] + a_ref[1]
    cnt = c_ref[0] + c_ref[1]
    denom = jnp.maximum(cnt, 1.0)
    hr = jnp.dot(h_ref[...], root_ref[...], preferred_element_type=_f32)
    hn = jnp.maximum(a / denom + hr + bias_ref[...], 0.0)
    hn_ref[...] = hn
    y_ref[...] = jnp.sum(hn * f2w_ref[...], axis=1, keepdims=True) + f2b_ref[...]


def _tc_update(aggr, cnt, h, root, bias1, f2w, f2b):
    return pl.pallas_call(
        _upd_body,
        out_shape=(
            jax.ShapeDtypeStruct((NP, W), _f32),
            jax.ShapeDtypeStruct((NP, 1), _f32),
        ),
    )(aggr, cnt, h, root, bias1, f2w, f2b)


def kernel(x, edge_index, edge_attr, fc1_w, fc1_b, kw1, kb1, kw2, kb2,
           kw3, kb3, root, bias, fc2_w, fc2_b):
    src = edge_index[0].astype(_i32)
    dst = edge_index[1].astype(_i32)
    pad = EP - E
    src_p = jnp.concatenate([src, jnp.zeros((pad,), _i32)])
    dst_p = jnp.concatenate([dst, jnp.full((pad,), DUMP, _i32)])
    dst3 = dst_p.reshape(NT, CH, 128)
    x_p = jnp.pad(x, (0, NP - N))

    zero_rt = jnp.zeros((RT, W), _f32)
    ones_blk = jnp.ones((128, W), _f32)

    xs, xd = _sc_endpoint_gather(x, src_p, dst_p)

    eat = jnp.concatenate(
        [jnp.pad(edge_attr, ((0, pad), (0, 0))).T, xs[None, :], xd[None, :]],
        axis=0)

    r2t = _tc_mlp(eat, kw1, kb1[:, None], kw2.astype(_bf16), kb2[:, None])

    h = _tc_init(x_p[:, None], fc1_w.T, fc1_b[None, :])

    cnt = _sc_degree_count(dst3, zero_rt, ones_blk)

    kw3b = kw3.astype(_bf16)
    kb3c = kb3[:, None]
    lanei = jnp.arange(KO, dtype=_i32)
    rsel = (lanei[:, None] // W == jnp.arange(W, dtype=_i32)[None, :]).astype(_f32)
    ssel = (lanei[None, :] % W == jnp.arange(W, dtype=_i32)[:, None]).astype(_f32)
    bias1 = bias[None, :]
    f2b = fc2_b[None, :]

    y = None
    for _ in range(DEPTH):
        xjt = _sc_gather_rows_t(h, src_p)
        msgt = _tc_msg(r2t, xjt, kw3b, kb3c, rsel, ssel)
        aggr = _sc_scatter_add_t(msgt, dst3, zero_rt)
        h, y = _tc_update(aggr, cnt, h, root, bias1, fc2_w, f2b)

    return y[:N, 0]


# speedup vs baseline: 6.2851x; 1.4572x over previous
import functools

import jax
import jax.numpy as jnp
from jax import lax
from jax.experimental import pallas as pl
from jax.experimental.pallas import tpu as pltpu
from jax.experimental.pallas import tpu_sc as plsc

N = 10000
E = 160000
W = 16
KW = 128
KO = W * W
DEPTH = 4

NC = 2
NS = 16
NT = NC * NS
L = 16

ET = 5120
CH = 40
CG = 1280
EP = NT * ET
NP = 10112
RT = NP // NS
DUMP = N

EB = 2048

_mesh = plsc.VectorSubcoreMesh(core_axis_name="c", subcore_axis_name="s")
_f32 = jnp.float32
_bf16 = jnp.bfloat16
_i32 = jnp.int32
_sc_params = pltpu.CompilerParams(
    needs_layout_passes=False, use_tc_tiling_on_sc=False)


def _wid():
    return lax.axis_index("c") * NS + lax.axis_index("s")


@functools.partial(
    pl.kernel,
    out_type=(jax.ShapeDtypeStruct((EP,), _f32), jax.ShapeDtypeStruct((EP,), _f32)),
    mesh=_mesh,
    compiler_params=_sc_params,
    scratch_types=[
        pltpu.VMEM((N,), _f32),
        pltpu.VMEM((ET,), _i32),
        pltpu.VMEM((ET,), _i32),
        pltpu.VMEM((ET,), _f32),
        pltpu.VMEM((ET,), _f32),
    ],
)
def _sc_endpoint_gather(x_hbm, src_hbm, dst_hbm, xs_hbm, xd_hbm,
                        x_v, s_v, d_v, xs_v, xd_v):
    base = _wid() * ET
    pltpu.sync_copy(x_hbm, x_v)
    pltpu.sync_copy(src_hbm.at[pl.ds(base, ET)], s_v)
    pltpu.sync_copy(dst_hbm.at[pl.ds(base, ET)], d_v)

    def body(i, carry):
        o = i * L
        xs_v[pl.ds(o, L)] = plsc.load_gather(x_v, [s_v[pl.ds(o, L)]])
        xd_v[pl.ds(o, L)] = plsc.load_gather(x_v, [d_v[pl.ds(o, L)]])
        return carry

    lax.fori_loop(0, ET // L, body, 0)
    pltpu.sync_copy(xs_v, xs_hbm.at[pl.ds(base, ET)])
    pltpu.sync_copy(xd_v, xd_hbm.at[pl.ds(base, ET)])


@functools.partial(
    pl.kernel,
    out_type=jax.ShapeDtypeStruct((W, EP), _f32),
    mesh=_mesh,
    compiler_params=_sc_params,
    scratch_types=[
        pltpu.VMEM((ET,), _i32),
        pltpu.VMEM((2, CG, W), _f32),
        pltpu.VMEM((W, ET), _f32),
        pltpu.SemaphoreType.DMA,
        pltpu.SemaphoreType.DMA,
    ],
)
def _sc_gather_rows_t(h_hbm, src_hbm, xjt_hbm, idx_v, rows_v, xjt_v, sem, osem):
    base = _wid() * ET
    pltpu.sync_copy(src_hbm.at[pl.ds(base, ET)], idx_v)
    lane = jnp.arange(L, dtype=_i32)
    nch = ET // CG
    pltpu.async_copy(h_hbm.at[idx_v.at[pl.ds(0, CG)]], rows_v.at[0], sem)

    def chunk(cix, carry):
        b = lax.rem(cix, 2)
        pltpu.make_async_copy(h_hbm.at[idx_v.at[pl.ds(0, CG)]],
                              rows_v.at[b], sem).wait()

        @pl.when(cix < nch - 1)
        def _():
            cb2 = (cix + 1) * CG
            pltpu.async_copy(h_hbm.at[idx_v.at[pl.ds(cb2, CG)]],
                             rows_v.at[1 - b], sem)

        cb = cix * CG
        bvec = jnp.broadcast_to(b, (L,)).astype(_i32)

        @plsc.parallel_loop(0, CG // L, unroll=2)
        def grp(g):
            rb = g * L
            ridx = lane + rb
            for i in range(W):
                vec = plsc.load_gather(
                    rows_v, [bvec, ridx, jnp.full((L,), i, _i32)])
                xjt_v[i, pl.ds(cb + rb, L)] = vec

        return carry

    lax.fori_loop(0, nch, chunk, 0)
    for i in range(W):
        pltpu.async_copy(xjt_v.at[i], xjt_hbm.at[i, pl.ds(base, ET)], osem)
    for i in range(W):
        pltpu.make_async_copy(xjt_v.at[i], xjt_hbm.at[i, pl.ds(base, ET)],
                              osem).wait()


@functools.partial(
    pl.kernel,
    out_type=jax.ShapeDtypeStruct((NC, NP, W), _f32),
    mesh=_mesh,
    compiler_params=_sc_params,
    scratch_types=[
        pltpu.VMEM((CH, 128), _i32),
        pltpu.VMEM((W, ET), _f32),
        pltpu.VMEM((2, 128, W), _f32),
        pltpu.VMEM((RT, W), _f32),
        pltpu.VMEM_SHARED((NP, W), _f32),
        pltpu.SemaphoreType.DMA,
        pltpu.SemaphoreType.DMA,
    ],
)
def _sc_scatter_add_t(msgt_hbm, dst3_hbm, zero_hbm, out_hbm,
                      idx_v, msgt_v, chunk_v, buf_v, aggr_s, isem, asem):
    c = lax.axis_index("c")
    s = lax.axis_index("s")
    w = c * NS + s
    base = w * ET
    pltpu.sync_copy(zero_hbm, buf_v)
    pltpu.sync_copy(buf_v, aggr_s.at[pl.ds(s * RT, RT)])
    pltpu.sync_copy(dst3_hbm.at[w], idx_v)
    for i in range(W):
        pltpu.async_copy(msgt_hbm.at[i, pl.ds(base, ET)], msgt_v.at[i], isem)
    for i in range(W):
        pltpu.make_async_copy(msgt_hbm.at[i, pl.ds(base, ET)], msgt_v.at[i],
                              isem).wait()
    plsc.subcore_barrier()
    lane = jnp.arange(L, dtype=_i32)

    def body(j, carry):
        jb = j * 128
        b = lax.rem(j, 2)
        bvec = jnp.broadcast_to(b, (L,)).astype(_i32)

        @plsc.parallel_loop(0, 128 // L, unroll=2)
        def grp(g):
            eb = g * L
            eidx = lane + eb
            for i in range(W):
                vec = msgt_v[i, pl.ds(jb + eb, L)]
                plsc.store_scatter(chunk_v,
                                   [bvec, eidx, jnp.full((L,), i, _i32)], vec)

        @pl.when(j > 0)
        def _():
            pltpu.make_async_copy(chunk_v.at[0], aggr_s.at[idx_v.at[0]],
                                  asem).wait()

        pltpu.async_copy(chunk_v.at[b], aggr_s.at[idx_v.at[j]], asem, add=True)
        return carry

    lax.fori_loop(0, CH, body, 0)
    pltpu.make_async_copy(chunk_v.at[0], aggr_s.at[idx_v.at[0]], asem).wait()
    plsc.subcore_barrier()
    pltpu.sync_copy(aggr_s.at[pl.ds(s * RT, RT)], buf_v)
    pltpu.sync_copy(buf_v, out_hbm.at[c, pl.ds(s * RT, RT)])


@functools.partial(
    pl.kernel,
    out_type=jax.ShapeDtypeStruct((NC, NP, W), _f32),
    mesh=_mesh,
    compiler_params=_sc_params,
    scratch_types=[
        pltpu.VMEM((CH, 128), _i32),
        pltpu.VMEM((128, W), _f32),
        pltpu.VMEM((RT, W), _f32),
        pltpu.VMEM_SHARED((NP, W), _f32),
    ],
)
def _sc_degree_count(dst3_hbm, zero_hbm, ones_hbm, out_hbm,
                     idx_v, one_v, buf_v, cnt_s):
    c = lax.axis_index("c")
    s = lax.axis_index("s")
    w = c * NS + s
    pltpu.sync_copy(zero_hbm, buf_v)
    pltpu.sync_copy(buf_v, cnt_s.at[pl.ds(s * RT, RT)])
    pltpu.sync_copy(ones_hbm, one_v)
    pltpu.sync_copy(dst3_hbm.at[w], idx_v)
    plsc.subcore_barrier()

    def body(j, carry):
        pltpu.sync_copy(one_v, cnt_s.at[idx_v.at[j]], add=True)
        return carry

    lax.fori_loop(0, CH, body, 0)
    plsc.subcore_barrier()
    pltpu.sync_copy(cnt_s.at[pl.ds(s * RT, RT)], buf_v)
    pltpu.sync_copy(buf_v, out_hbm.at[c, pl.ds(s * RT, RT)])


def _mlp_body(eat_ref, kw1_ref, kb1_ref, kw2_ref, kb2_ref, o_ref):
    k1 = jnp.dot(kw1_ref[...], eat_ref[...], preferred_element_type=_f32)
    k1 = jnp.maximum(k1 + kb1_ref[...], 0.0).astype(_bf16)
    k2 = jnp.dot(kw2_ref[...], k1, preferred_element_type=_f32) + kb2_ref[...]
    o_ref[...] = jnp.maximum(k2, 0.0).astype(_bf16)


def _tc_mlp(eat, kw1, kb1c, kw2b, kb2c):
    grid = (EP // EB,)
    return pl.pallas_call(
        _mlp_body,
        grid=grid,
        in_specs=[
            pl.BlockSpec((4, EB), lambda i: (0, i)),
            pl.BlockSpec((KW, 4), lambda i: (0, 0)),
            pl.BlockSpec((KW, 1), lambda i: (0, 0)),
            pl.BlockSpec((KW, KW), lambda i: (0, 0)),
            pl.BlockSpec((KW, 1), lambda i: (0, 0)),
        ],
        out_specs=pl.BlockSpec((KW, EB), lambda i: (0, i)),
        out_shape=jax.ShapeDtypeStruct((KW, EP), _bf16),
    )(eat, kw1, kb1c, kw2b, kb2c)


def _msg_body(r2t_ref, xjt_ref, kw3_ref, kb3_ref, rsel_ref, ssel_ref, o_ref):
    k = jnp.dot(kw3_ref[...], r2t_ref[...], preferred_element_type=_f32)
    k = k + kb3_ref[...]
    xrep = jnp.dot(rsel_ref[...], xjt_ref[...], preferred_element_type=_f32)
    o_ref[...] = jnp.dot(ssel_ref[...], xrep * k, preferred_element_type=_f32)


def _tc_msg(r2t, xjt, kw3b, kb3c, rsel, ssel):
    grid = (EP // EB,)
    return pl.pallas_call(
        _msg_body,
        grid=grid,
        in_specs=[
            pl.BlockSpec((KW, EB), lambda i: (0, i)),
            pl.BlockSpec((W, EB), lambda i: (0, i)),
            pl.BlockSpec((KO, KW), lambda i: (0, 0)),
            pl.BlockSpec((KO, 1), lambda i: (0, 0)),
            pl.BlockSpec((KO, W), lambda i: (0, 0)),
            pl.BlockSpec((W, KO), lambda i: (0, 0)),
        ],
        out_specs=pl.BlockSpec((W, EB), lambda i: (0, i)),
        out_shape=jax.ShapeDtypeStruct((W, EP), _f32),
    )(r2t, xjt, kw3b, kb3c, rsel, ssel)


def _init_body(x_ref, w_ref, b_ref, o_ref):
    o_ref[...] = x_ref[...] * w_ref[...] + b_ref[...]


def _tc_init(xp1, w, b):
    return pl.pallas_call(
        _init_body,
        out_shape=jax.ShapeDtypeStruct((NP, W), _f32),
    )(xp1, w, b)


def _upd_body(a_ref, c_ref, h_ref, root_ref, bias_ref, f2w_ref, f2b_ref,
              hn_ref, y_ref):
    a = a_ref[0] + a_ref[1]
    cnt = c_ref[0] + c_ref[1]
    denom = jnp.maximum(cnt, 1.0)
    hr = jnp.dot(h_ref[...], root_ref[...], preferred_element_type=_f32)
    hn = jnp.maximum(a / denom + hr + bias_ref[...], 0.0)
    hn_ref[...] = hn
    y_ref[...] = jnp.sum(hn * f2w_ref[...], axis=1, keepdims=True) + f2b_ref[...]


def _tc_update(aggr, cnt, h, root, bias1, f2w, f2b):
    return pl.pallas_call(
        _upd_body,
        out_shape=(
            jax.ShapeDtypeStruct((NP, W), _f32),
            jax.ShapeDtypeStruct((NP, 1), _f32),
        ),
    )(aggr, cnt, h, root, bias1, f2w, f2b)


def kernel(x, edge_index, edge_attr, fc1_w, fc1_b, kw1, kb1, kw2, kb2,
           kw3, kb3, root, bias, fc2_w, fc2_b):
    src = edge_index[0].astype(_i32)
    dst = edge_index[1].astype(_i32)
    pad = EP - E
    src_p = jnp.concatenate([src, jnp.zeros((pad,), _i32)])
    dst_p = jnp.concatenate([dst, jnp.full((pad,), DUMP, _i32)])
    dst3 = dst_p.reshape(NT, CH, 128)
    x_p = jnp.pad(x, (0, NP - N))

    zero_rt = jnp.zeros((RT, W), _f32)
    ones_blk = jnp.ones((128, W), _f32)

    xs, xd = _sc_endpoint_gather(x, src_p, dst_p)

    eat = jnp.concatenate(
        [jnp.pad(edge_attr, ((0, pad), (0, 0))).T, xs[None, :], xd[None, :]],
        axis=0)

    r2t = _tc_mlp(eat, kw1, kb1[:, None], kw2.astype(_bf16), kb2[:, None])

    h = _tc_init(x_p[:, None], fc1_w.T, fc1_b[None, :])

    cnt = _sc_degree_count(dst3, zero_rt, ones_blk)

    kw3b = kw3.astype(_bf16)
    kb3c = kb3[:, None]
    lanei = jnp.arange(KO, dtype=_i32)
    rsel = (lanei[:, None] // W == jnp.arange(W, dtype=_i32)[None, :]).astype(_f32)
    ssel = (lanei[None, :] % W == jnp.arange(W, dtype=_i32)[:, None]).astype(_f32)
    bias1 = bias[None, :]
    f2b = fc2_b[None, :]

    y = None
    for _ in range(DEPTH):
        xjt = _sc_gather_rows_t(h, src_p)
        msgt = _tc_msg(r2t, xjt, kw3b, kb3c, rsel, ssel)
        aggr = _sc_scatter_add_t(msgt, dst3, zero_rt)
        h, y = _tc_update(aggr, cnt, h, root, bias1, fc2_w, f2b)

    return y[:N, 0]
